# baseline (speedup 1.0000x reference)
import functools

import jax
import jax.numpy as jnp
from jax import lax
from jax.experimental import pallas as pl
from jax.experimental.pallas import tpu as pltpu
from jax.experimental.pallas import tpu_sc as plsc

N_ROWS = 1_000_000
DIM = 32
BATCH = 16384

_info = plsc.get_sparse_core_info()
_NC, _NS = _info.num_cores, _info.num_subcores
_NW = _NC * _NS
_B_PER_W = BATCH // _NW
_G = 8
_NGROUPS = _B_PER_W // _G
_NBANKS = 3


def _gather_body(x_hbm, tableT_hbm, outT_hbm, idx_v, blocks_v, cols_v, sem):
    wid = lax.axis_index("s") * _NC + lax.axis_index("c")
    base = wid * _B_PER_W
    pltpu.sync_copy(x_hbm.at[pl.ds(base, _B_PER_W)], idx_v.at[pl.ds(0, _B_PER_W)])

    iota = lax.broadcasted_iota(jnp.int32, (16,), 0)

    def _fire(idx16, l, bank):
        i = idx16[l]
        col0 = pl.multiple_of((i // 128) * 128, 128)
        pltpu.async_copy(
            tableT_hbm.at[:, pl.ds(col0, 128)],
            blocks_v.at[bank, l],
            sem,
        )

    idx16_0 = idx_v[pl.ds(0, 16)]
    idx16_1 = idx_v[pl.ds(_G, 16)]
    for l in range(_G):
        _fire(idx16_0, l, 0)
    for l in range(_G):
        _fire(idx16_1, l, 1)

    def _step(g, carry):
        idx16_cur, idx16_next = carry
        bank = lax.rem(g, _NBANKS)
        idx16_fire = idx_v[pl.ds((g + 2) * _G, 16)]

        @pl.when(g < _NGROUPS - 2)
        def _():
            fire_bank = lax.rem(g + 2, _NBANKS)
            for l in range(_G):
                _fire(idx16_fire, l, fire_bank)

        for l in range(_G):
            pltpu.make_async_copy(
                tableT_hbm.at[:, pl.ds(0, 128)],
                blocks_v.at[bank, l],
                sem,
            ).wait()

        for l in range(_G):
            r = lax.rem(idx16_cur[l], 128)
            rvec = lax.broadcast(r, (16,))
            b = g * _G + l
            bvec = lax.broadcast(b, (16,))
            for h in range(2):
                rows = iota + 16 * h
                vals = plsc.load_gather(blocks_v.at[bank, l], [rows, rvec])
                plsc.store_scatter(cols_v, [rows, bvec], vals)
        return (idx16_next, idx16_fire)

    lax.fori_loop(0, _NGROUPS, _step, (idx16_0, idx16_1), unroll=False)
    pltpu.sync_copy(cols_v, outT_hbm.at[:, pl.ds(base, _B_PER_W)])


@jax.jit
def _gather(x, tableT):
    mesh = plsc.VectorSubcoreMesh(core_axis_name="c", subcore_axis_name="s")
    kern = functools.partial(
        pl.kernel,
        mesh=mesh,
        out_type=jax.ShapeDtypeStruct((DIM, BATCH), jnp.float32),
        scratch_types=[
            pltpu.VMEM((_B_PER_W + 2 * 16, ), jnp.int32),
            pltpu.VMEM((_NBANKS, _G, DIM, 128), jnp.float32),
            pltpu.VMEM((DIM, _B_PER_W), jnp.float32),
            pltpu.SemaphoreType.DMA,
        ],
        compiler_params=pltpu.CompilerParams(needs_layout_passes=False),
    )(_gather_body)
    return kern(x, tableT)


def kernel(x, table):
    return _gather(x, table.T).T
